# baseline (speedup 1.0000x reference)
import functools

import jax
import jax.numpy as jnp
from jax.experimental import pallas as pl
from jax.experimental.pallas import tpu as pltpu

N = 1024
B = 32
U = 64
C = 2 * U
BLK = 256
F32 = jnp.float32
BF16 = jnp.bfloat16

_ARB2 = pltpu.CompilerParams(dimension_semantics=("arbitrary", "arbitrary"))


def _unpack_nm(h2, blkp):
    v = h2.astype(BF16).reshape(B, blkp // 2, 2 * U)
    w = jnp.stack([v[:, :, :U], v[:, :, U:]], axis=2)
    return jnp.transpose(w.reshape(B, blkp, U), (1, 0, 2))


def _prep_body(x_ref, hs_ref, y0_ref, h1p_ref):
    x = x_ref[...]
    blkp = x.shape[0]
    hs = hs_ref[...]
    h0 = _unpack_nm(hs[0], blkp)
    h1 = _unpack_nm(hs[1], blkp)
    y0 = jnp.concatenate([x, h0], axis=-1)
    y0_ref[...] = y0.reshape(blkp, B * C)
    h1p = jnp.concatenate([jnp.zeros_like(x), h1], axis=-1)
    h1p_ref[...] = h1p.reshape(blkp, B * C)


def _prep(xin, hs):
    blkp = 128
    return pl.pallas_call(
        _prep_body,
        grid=(N // blkp,),
        in_specs=[
            pl.BlockSpec((blkp, B, U), lambda i: (i, 0, 0)),
            pl.BlockSpec((2, B, blkp * U), lambda i: (0, 0, i)),
        ],
        out_specs=[
            pl.BlockSpec((blkp, B * C), lambda i: (i, 0)),
            pl.BlockSpec((blkp, B * C), lambda i: (i, 0)),
        ],
        out_shape=[
            jax.ShapeDtypeStruct((N, B * C), BF16),
            jax.ShapeDtypeStruct((N, B * C), BF16),
        ],
    )(xin, hs)


def _gate_body(a_ref, y0_ref, wg_ref, wcx_ref, bg_ref, s0p_ref, p_ref, y1_scr):
    ph = pl.program_id(0)
    i = pl.program_id(1)
    a = a_ref[...]

    @pl.when(ph == 0)
    def _():
        y1_scr[pl.ds(i * BLK, BLK), :] = jnp.dot(
            a, y0_ref[...], preferred_element_type=F32).astype(BF16)

    @pl.when(ph == 1)
    def _():
        rows = pl.ds(i * BLK, BLK)
        y0b = y0_ref[rows, :]
        y1b = y1_scr[rows, :]
        y2 = 2.0 * jnp.dot(a, y1_scr[...], preferred_element_type=F32) - y0b.astype(F32)
        y0r = y0b.reshape(BLK * B, C)
        y1r = y1b.reshape(BLK * B, C)
        y2r = y2.astype(BF16).reshape(BLK * B, C)
        g = jnp.dot(y0r, wg_ref[0], preferred_element_type=F32)
        g += jnp.dot(y1r, wg_ref[1], preferred_element_type=F32)
        g += jnp.dot(y2r, wg_ref[2], preferred_element_type=F32)
        g = jax.nn.sigmoid((g + bg_ref[...]).astype(BF16))
        cx = jnp.dot(y0r, wcx_ref[0], preferred_element_type=F32)
        cx += jnp.dot(y1r, wcx_ref[1], preferred_element_type=F32)
        cx += jnp.dot(y2r, wcx_ref[2], preferred_element_type=F32)
        r = g[:, :U]
        u = g[:, U:]
        hx = y0r[:, U:]
        s0 = r * hx
        s0p_ref[...] = jnp.concatenate([s0, jnp.zeros_like(s0)], axis=-1).reshape(BLK, B * C)
        p_ref[...] = jnp.concatenate(
            [cx.astype(BF16), u], axis=-1).reshape(BLK, B * C)


def _gate(adj, y0, wg, wcx, bg):
    return pl.pallas_call(
        _gate_body,
        grid=(2, N // BLK),
        in_specs=[
            pl.BlockSpec((BLK, N), lambda p, i: (i, 0)),
            pl.BlockSpec((N, B * C), lambda p, i: (0, 0)),
            pl.BlockSpec((3, C, C), lambda p, i: (0, 0, 0)),
            pl.BlockSpec((3, C, U), lambda p, i: (0, 0, 0)),
            pl.BlockSpec((1, C), lambda p, i: (0, 0)),
        ],
        out_specs=[
            pl.BlockSpec((BLK, B * C), lambda p, i: (jnp.where(p == 1, i, 0), 0)),
            pl.BlockSpec((BLK, B * C), lambda p, i: (jnp.where(p == 1, i, 0), 0)),
        ],
        out_shape=[
            jax.ShapeDtypeStruct((N, B * C), BF16),
            jax.ShapeDtypeStruct((N, B * C), BF16),
        ],
        scratch_shapes=[pltpu.VMEM((N, B * C), BF16)],
        compiler_params=_ARB2,
    )(adj, y0, wg, wcx, bg)


def _cand_body(is_l0, *refs):
    (a_ref, s0p_ref, p_ref, y0_ref, wcs_ref, bc_ref) = refs[:6]
    if is_l0:
        h1p_ref, y0n_ref, stack_ref = refs[6:9]
        out1_ref = None
        s1_scr = refs[9]
    else:
        _, stack_ref, out1_ref = refs[6:9]
        h1p_ref = y0n_ref = None
        s1_scr = refs[9]
    ph = pl.program_id(0)
    i = pl.program_id(1)
    a = a_ref[...]

    @pl.when(ph == 0)
    def _():
        s1_scr[pl.ds(i * BLK, BLK), :] = jnp.dot(
            a, s0p_ref[...], preferred_element_type=F32).astype(BF16)

    @pl.when(ph == 1)
    def _():
        rows = pl.ds(i * BLK, BLK)
        s0b = s0p_ref[rows, :]
        s1b = s1_scr[rows, :]
        s2 = 2.0 * jnp.dot(a, s1_scr[...], preferred_element_type=F32) - s0b.astype(F32)
        s0r = s0b.reshape(BLK * B, C)
        s1r = s1b.reshape(BLK * B, C)
        s2r = s2.astype(BF16).reshape(BLK * B, C)
        pr = p_ref[...].reshape(BLK * B, C)
        c = jnp.dot(s0r, wcs_ref[0], preferred_element_type=F32)
        c += jnp.dot(s1r, wcs_ref[1], preferred_element_type=F32)
        c += jnp.dot(s2r, wcs_ref[2], preferred_element_type=F32)
        c = jnp.tanh((c + pr[:, :U].astype(F32) + bc_ref[...]).astype(BF16)).astype(F32)
        u = pr[:, U:].astype(F32)
        y0r = y0_ref[...].reshape(BLK * B, C)
        hx = y0r[:, U:].astype(F32)
        hn = u * hx + (1.0 - u) * c

        if y0n_ref is not None:
            h1pr = h1p_ref[...].reshape(BLK * B, C)
            y0n = jnp.concatenate([hn.astype(BF16), h1pr[:, U:]], axis=-1)
            y0n_ref[...] = y0n.reshape(BLK, B * C)

        hp = hn.reshape(BLK // 2, 2, B, U)
        cc = jnp.concatenate([hp[:, 0], hp[:, 1]], axis=-1)
        hbm = jnp.transpose(cc, (1, 0, 2)).reshape(B, BLK * U)
        stack_ref[...] = hbm.reshape(1, B, BLK * U)
        if out1_ref is not None:
            out1_ref[...] = hbm


def _cand(adj, s0p, p, y0, wcs, bc, h1p, stack_in):
    is_l0 = stack_in is None

    def blk1(p_, i_):
        return (jnp.where(p_ == 1, i_, 0), 0)

    in_specs = [
        pl.BlockSpec((BLK, N), lambda p_, i_: (i_, 0)),
        pl.BlockSpec((N, B * C), lambda p_, i_: (0, 0)),
        pl.BlockSpec((BLK, B * C), blk1),
        pl.BlockSpec((BLK, B * C), blk1),
        pl.BlockSpec((3, C, U), lambda p_, i_: (0, 0, 0)),
        pl.BlockSpec((1, U), lambda p_, i_: (0, 0)),
    ]
    args = [adj, s0p, p, y0, wcs, bc]
    slot = 0 if is_l0 else 1
    stack_spec = pl.BlockSpec(
        (1, B, BLK * U), lambda p_, i_: (slot, 0, jnp.where(p_ == 1, i_, 0)))
    stack_shape = jax.ShapeDtypeStruct((2, B, N * U), F32)
    aliases = {}
    if is_l0:
        in_specs.append(pl.BlockSpec((BLK, B * C), blk1))
        args.append(h1p)
        out_specs = [pl.BlockSpec((BLK, B * C), blk1), stack_spec]
        out_shape = [jax.ShapeDtypeStruct((N, B * C), BF16), stack_shape]
    else:
        in_specs.append(pl.BlockSpec((1, 8, 128), lambda p_, i_: (0, 0, 0)))
        args.append(stack_in)
        aliases = {6: 0}
        out_specs = [
            stack_spec,
            pl.BlockSpec((B, BLK * U), lambda p_, i_: (0, jnp.where(p_ == 1, i_, 0))),
        ]
        out_shape = [stack_shape, jax.ShapeDtypeStruct((B, N * U), F32)]
    return pl.pallas_call(
        functools.partial(_cand_body, is_l0),
        grid=(2, N // BLK),
        in_specs=in_specs,
        out_specs=out_specs,
        out_shape=out_shape,
        input_output_aliases=aliases,
        scratch_shapes=[pltpu.VMEM((N, B * C), BF16)],
        compiler_params=_ARB2,
    )(*args)


def _layer(adj_bf, y0, h1p, wg, bg, wcx, wcs, bc, stack_in=None):
    s0p, p = _gate(adj_bf, y0, wg, wcx, bg)
    return _cand(adj_bf, s0p, p, y0, wcs, bc, h1p, stack_in)


def _prep_w(w, cin):
    o = w.shape[1]
    wr = w.reshape(cin + U, 3, o).transpose(1, 0, 2)
    wx = wr[:, :cin, :]
    wh = wr[:, cin:, :]
    pad = jnp.zeros((3, U - cin, o), w.dtype)
    wxp = jnp.concatenate([wx, pad], axis=1)
    return wxp, wh


def kernel(inputs, adj, hidden_state,
           W_gate_0, b_gate_0, W_cand_0, b_cand_0,
           W_gate_1, b_gate_1, W_cand_1, b_cand_1):
    adj_bf = adj.astype(BF16)
    xin = inputs.astype(BF16).reshape(B, N, 2).transpose(1, 0, 2)
    xin = jnp.pad(xin, ((0, 0), (0, 0), (0, U - 2)))
    y0_l0, h1p = _prep(xin, hidden_state)

    zU = jnp.zeros((3, U, U), F32)
    wgx0, wgh0 = _prep_w(W_gate_0, 2)
    wg0 = jnp.concatenate([wgx0, wgh0], axis=1).astype(BF16)
    wcx0, wcs0 = _prep_w(W_cand_0, 2)
    wcx0 = jnp.concatenate([wcx0, zU], axis=1).astype(BF16)
    wcs0 = jnp.concatenate([wcs0, zU], axis=1).astype(BF16)
    wgx1, wgh1 = _prep_w(W_gate_1, U)
    wg1 = jnp.concatenate([wgx1, wgh1], axis=1).astype(BF16)
    wcx1, wcs1 = _prep_w(W_cand_1, U)
    wcx1 = jnp.concatenate([wcx1, zU], axis=1).astype(BF16)
    wcs1 = jnp.concatenate([wcs1, zU], axis=1).astype(BF16)
    bg0 = b_gate_0.reshape(1, C)
    bc0 = b_cand_0.reshape(1, U)
    bg1 = b_gate_1.reshape(1, C)
    bc1 = b_cand_1.reshape(1, U)

    y0_l1, stack0 = _layer(adj_bf, y0_l0, h1p, wg0, bg0, wcx0, wcs0, bc0)
    stack, out1 = _layer(adj_bf, y0_l1, h1p, wg1, bg1, wcx1, wcs1, bc1,
                         stack_in=stack0)

    return (out1, stack)


# speedup vs baseline: 1.0730x; 1.0730x over previous
import functools

import jax
import jax.numpy as jnp
import numpy as np
from jax.experimental import pallas as pl
from jax.experimental.pallas import tpu as pltpu

N = 1024
B = 32
U = 64
C = 2 * U
BLK = 256
F32 = jnp.float32
BF16 = jnp.bfloat16

_ARB2 = pltpu.CompilerParams(dimension_semantics=("arbitrary", "arbitrary"))

_PIV = np.arange(N).reshape(N // 128, 64, 2).transpose(0, 2, 1).reshape(-1)


def _unpack_nm(h2, blkp):
    v = h2.astype(BF16).reshape(B, blkp // 2, 2 * U)
    t = jnp.transpose(v, (1, 0, 2))
    return jnp.concatenate([t[:, :, :U], t[:, :, U:]], axis=0)


def _prep_body(x_ref, hs_ref, y0_ref, h1p_ref):
    x = x_ref[...]
    blkp = x.shape[0]
    hs = hs_ref[...]
    h0 = _unpack_nm(hs[0], blkp)
    h1 = _unpack_nm(hs[1], blkp)
    y0 = jnp.concatenate([x, h0], axis=-1)
    y0_ref[...] = y0.reshape(blkp, B * C)
    h1p = jnp.concatenate([jnp.zeros_like(x), h1], axis=-1)
    h1p_ref[...] = h1p.reshape(blkp, B * C)


def _prep(xin, hs):
    blkp = 128
    return pl.pallas_call(
        _prep_body,
        grid=(N // blkp,),
        in_specs=[
            pl.BlockSpec((blkp, B, U), lambda i: (i, 0, 0)),
            pl.BlockSpec((2, B, blkp * U), lambda i: (0, 0, i)),
        ],
        out_specs=[
            pl.BlockSpec((blkp, B * C), lambda i: (i, 0)),
            pl.BlockSpec((blkp, B * C), lambda i: (i, 0)),
        ],
        out_shape=[
            jax.ShapeDtypeStruct((N, B * C), BF16),
            jax.ShapeDtypeStruct((N, B * C), BF16),
        ],
    )(xin, hs)


def _gate_body(a_ref, y0_ref, wg_ref, wcx_ref, bg_ref, s0p_ref, p_ref, y1_scr):
    ph = pl.program_id(0)
    i = pl.program_id(1)
    a = a_ref[...]

    @pl.when(ph == 0)
    def _():
        y1_scr[pl.ds(i * BLK, BLK), :] = jnp.dot(
            a, y0_ref[...], preferred_element_type=F32).astype(BF16)

    @pl.when(ph == 1)
    def _():
        rows = pl.ds(i * BLK, BLK)
        y0b = y0_ref[rows, :]
        y1b = y1_scr[rows, :]
        y2 = 2.0 * jnp.dot(a, y1_scr[...], preferred_element_type=F32) - y0b.astype(F32)
        y0r = y0b.reshape(BLK * B, C)
        y1r = y1b.reshape(BLK * B, C)
        y2r = y2.astype(BF16).reshape(BLK * B, C)
        g = jnp.dot(y0r, wg_ref[0], preferred_element_type=F32)
        g += jnp.dot(y1r, wg_ref[1], preferred_element_type=F32)
        g += jnp.dot(y2r, wg_ref[2], preferred_element_type=F32)
        g = jax.nn.sigmoid((g + bg_ref[...]).astype(BF16))
        cx = jnp.dot(y0r, wcx_ref[0], preferred_element_type=F32)
        cx += jnp.dot(y1r, wcx_ref[1], preferred_element_type=F32)
        cx += jnp.dot(y2r, wcx_ref[2], preferred_element_type=F32)
        r = g[:, :U]
        u = g[:, U:]
        hx = y0r[:, U:]
        s0 = r * hx
        s0p_ref[...] = jnp.concatenate([s0, jnp.zeros_like(s0)], axis=-1).reshape(BLK, B * C)
        p_ref[...] = jnp.concatenate(
            [cx.astype(BF16), u], axis=-1).reshape(BLK, B * C)


def _gate(adj, y0, wg, wcx, bg):
    return pl.pallas_call(
        _gate_body,
        grid=(2, N // BLK),
        in_specs=[
            pl.BlockSpec((BLK, N), lambda p, i: (i, 0)),
            pl.BlockSpec((N, B * C), lambda p, i: (0, 0)),
            pl.BlockSpec((3, C, C), lambda p, i: (0, 0, 0)),
            pl.BlockSpec((3, C, U), lambda p, i: (0, 0, 0)),
            pl.BlockSpec((1, C), lambda p, i: (0, 0)),
        ],
        out_specs=[
            pl.BlockSpec((BLK, B * C), lambda p, i: (jnp.where(p == 1, i, 0), 0)),
            pl.BlockSpec((BLK, B * C), lambda p, i: (jnp.where(p == 1, i, 0), 0)),
        ],
        out_shape=[
            jax.ShapeDtypeStruct((N, B * C), BF16),
            jax.ShapeDtypeStruct((N, B * C), BF16),
        ],
        scratch_shapes=[pltpu.VMEM((N, B * C), BF16)],
        compiler_params=_ARB2,
    )(adj, y0, wg, wcx, bg)


def _cand_body(is_l0, *refs):
    (a_ref, s0p_ref, p_ref, y0_ref, wcs_ref, bc_ref) = refs[:6]
    if is_l0:
        h1p_ref, y0n_ref, stack_ref = refs[6:9]
        out1_ref = None
        s1_scr = refs[9]
    else:
        _, stack_ref, out1_ref = refs[6:9]
        h1p_ref = y0n_ref = None
        s1_scr = refs[9]
    ph = pl.program_id(0)
    i = pl.program_id(1)
    a = a_ref[...]

    @pl.when(ph == 0)
    def _():
        s1_scr[pl.ds(i * BLK, BLK), :] = jnp.dot(
            a, s0p_ref[...], preferred_element_type=F32).astype(BF16)

    @pl.when(ph == 1)
    def _():
        rows = pl.ds(i * BLK, BLK)
        s0b = s0p_ref[rows, :]
        s1b = s1_scr[rows, :]
        s2 = 2.0 * jnp.dot(a, s1_scr[...], preferred_element_type=F32) - s0b.astype(F32)
        s0r = s0b.reshape(BLK * B, C)
        s1r = s1b.reshape(BLK * B, C)
        s2r = s2.astype(BF16).reshape(BLK * B, C)
        pr = p_ref[...].reshape(BLK * B, C)
        c = jnp.dot(s0r, wcs_ref[0], preferred_element_type=F32)
        c += jnp.dot(s1r, wcs_ref[1], preferred_element_type=F32)
        c += jnp.dot(s2r, wcs_ref[2], preferred_element_type=F32)
        c = jnp.tanh((c + pr[:, :U].astype(F32) + bc_ref[...]).astype(BF16)).astype(F32)
        u = pr[:, U:].astype(F32)
        y0r = y0_ref[...].reshape(BLK * B, C)
        hx = y0r[:, U:].astype(F32)
        hn = u * hx + (1.0 - u) * c

        if y0n_ref is not None:
            h1pr = h1p_ref[...].reshape(BLK * B, C)
            y0n = jnp.concatenate([hn.astype(BF16), h1pr[:, U:]], axis=-1)
            y0n_ref[...] = y0n.reshape(BLK, B * C)

        h5 = hn.reshape(BLK // 128, 2, 64, B, U)
        cc = jnp.concatenate([h5[:, 0], h5[:, 1]], axis=-1)
        hbm = jnp.transpose(cc, (2, 0, 1, 3)).reshape(B, BLK * U)
        stack_ref[...] = hbm.reshape(1, B, BLK * U)
        if out1_ref is not None:
            out1_ref[...] = hbm


def _cand(adj, s0p, p, y0, wcs, bc, h1p, stack_in):
    is_l0 = stack_in is None

    def blk1(p_, i_):
        return (jnp.where(p_ == 1, i_, 0), 0)

    in_specs = [
        pl.BlockSpec((BLK, N), lambda p_, i_: (i_, 0)),
        pl.BlockSpec((N, B * C), lambda p_, i_: (0, 0)),
        pl.BlockSpec((BLK, B * C), blk1),
        pl.BlockSpec((BLK, B * C), blk1),
        pl.BlockSpec((3, C, U), lambda p_, i_: (0, 0, 0)),
        pl.BlockSpec((1, U), lambda p_, i_: (0, 0)),
    ]
    args = [adj, s0p, p, y0, wcs, bc]
    slot = 0 if is_l0 else 1
    stack_spec = pl.BlockSpec(
        (1, B, BLK * U), lambda p_, i_: (slot, 0, jnp.where(p_ == 1, i_, 0)))
    stack_shape = jax.ShapeDtypeStruct((2, B, N * U), F32)
    aliases = {}
    if is_l0:
        in_specs.append(pl.BlockSpec((BLK, B * C), blk1))
        args.append(h1p)
        out_specs = [pl.BlockSpec((BLK, B * C), blk1), stack_spec]
        out_shape = [jax.ShapeDtypeStruct((N, B * C), BF16), stack_shape]
    else:
        in_specs.append(pl.BlockSpec((1, 8, 128), lambda p_, i_: (0, 0, 0)))
        args.append(stack_in)
        aliases = {6: 0}
        out_specs = [
            stack_spec,
            pl.BlockSpec((B, BLK * U), lambda p_, i_: (0, jnp.where(p_ == 1, i_, 0))),
        ]
        out_shape = [stack_shape, jax.ShapeDtypeStruct((B, N * U), F32)]
    return pl.pallas_call(
        functools.partial(_cand_body, is_l0),
        grid=(2, N // BLK),
        in_specs=in_specs,
        out_specs=out_specs,
        out_shape=out_shape,
        input_output_aliases=aliases,
        scratch_shapes=[pltpu.VMEM((N, B * C), BF16)],
        compiler_params=_ARB2,
    )(*args)


def _layer(adj_bf, y0, h1p, wg, bg, wcx, wcs, bc, stack_in=None):
    s0p, p = _gate(adj_bf, y0, wg, wcx, bg)
    return _cand(adj_bf, s0p, p, y0, wcs, bc, h1p, stack_in)


def _prep_w(w, cin):
    o = w.shape[1]
    wr = w.reshape(cin + U, 3, o).transpose(1, 0, 2)
    wx = wr[:, :cin, :]
    wh = wr[:, cin:, :]
    pad = jnp.zeros((3, U - cin, o), w.dtype)
    wxp = jnp.concatenate([wx, pad], axis=1)
    return wxp, wh


def kernel(inputs, adj, hidden_state,
           W_gate_0, b_gate_0, W_cand_0, b_cand_0,
           W_gate_1, b_gate_1, W_cand_1, b_cand_1):
    adj_bf = adj[_PIV][:, _PIV].astype(BF16)
    xin = inputs.astype(BF16).reshape(B, N, 2).transpose(1, 0, 2)[_PIV]
    xin = jnp.pad(xin, ((0, 0), (0, 0), (0, U - 2)))
    y0_l0, h1p = _prep(xin, hidden_state)

    zU = jnp.zeros((3, U, U), F32)
    wgx0, wgh0 = _prep_w(W_gate_0, 2)
    wg0 = jnp.concatenate([wgx0, wgh0], axis=1).astype(BF16)
    wcx0, wcs0 = _prep_w(W_cand_0, 2)
    wcx0 = jnp.concatenate([wcx0, zU], axis=1).astype(BF16)
    wcs0 = jnp.concatenate([wcs0, zU], axis=1).astype(BF16)
    wgx1, wgh1 = _prep_w(W_gate_1, U)
    wg1 = jnp.concatenate([wgx1, wgh1], axis=1).astype(BF16)
    wcx1, wcs1 = _prep_w(W_cand_1, U)
    wcx1 = jnp.concatenate([wcx1, zU], axis=1).astype(BF16)
    wcs1 = jnp.concatenate([wcs1, zU], axis=1).astype(BF16)
    bg0 = b_gate_0.reshape(1, C)
    bc0 = b_cand_0.reshape(1, U)
    bg1 = b_gate_1.reshape(1, C)
    bc1 = b_cand_1.reshape(1, U)

    y0_l1, stack0 = _layer(adj_bf, y0_l0, h1p, wg0, bg0, wcx0, wcs0, bc0)
    stack, out1 = _layer(adj_bf, y0_l1, h1p, wg1, bg1, wcx1, wcs1, bc1,
                         stack_in=stack0)

    return (out1, stack)


# speedup vs baseline: 1.1147x; 1.0389x over previous
import functools

import jax
import jax.numpy as jnp
import numpy as np
from jax.experimental import pallas as pl
from jax.experimental.pallas import tpu as pltpu

N = 1024
B = 32
U = 64
C = 2 * U
BLK = 256
F32 = jnp.float32
BF16 = jnp.bfloat16

_ARB2 = pltpu.CompilerParams(dimension_semantics=("arbitrary", "arbitrary"))

_PIV = np.arange(N).reshape(N // 128, 64, 2).transpose(0, 2, 1).reshape(-1)


def _unpack_nm(h2, blkp):
    v = h2.astype(BF16).reshape(B, blkp // 2, 2 * U)
    t = jnp.transpose(v, (1, 0, 2))
    return jnp.concatenate([t[:, :, :U], t[:, :, U:]], axis=0)


def _prep_body(x_ref, hs_ref, y0_ref, h1p_ref):
    x = x_ref[...]
    blkp = x.shape[0]
    hs = hs_ref[...]
    h0 = _unpack_nm(hs[0], blkp)
    h1 = _unpack_nm(hs[1], blkp)
    y0 = jnp.concatenate([x, h0], axis=-1)
    y0_ref[...] = y0.reshape(blkp, B * C)
    h1p = jnp.concatenate([jnp.zeros_like(x), h1], axis=-1)
    h1p_ref[...] = h1p.reshape(blkp, B * C)


def _prep(xin, hs):
    blkp = 128
    return pl.pallas_call(
        _prep_body,
        grid=(N // blkp,),
        in_specs=[
            pl.BlockSpec((blkp, B, U), lambda i: (i, 0, 0)),
            pl.BlockSpec((2, B, blkp * U), lambda i: (0, 0, i)),
        ],
        out_specs=[
            pl.BlockSpec((blkp, B * C), lambda i: (i, 0)),
            pl.BlockSpec((blkp, B * C), lambda i: (i, 0)),
        ],
        out_shape=[
            jax.ShapeDtypeStruct((N, B * C), BF16),
            jax.ShapeDtypeStruct((N, B * C), BF16),
        ],
    )(xin, hs)


_LBLK = 128


def _layer_body(is_l0, *refs):
    (a_ref, y0_ref, wg_ref, wcx_ref, wcs_ref, bg_ref, bc_ref) = refs[:7]
    if is_l0:
        h1p_ref, y0n_ref, stack_ref = refs[7:10]
        out1_ref = None
        y1_scr, sp0_scr, sp1_scr, u_scr = refs[10:]
    else:
        _, stack_ref, out1_ref = refs[7:10]
        h1p_ref = y0n_ref = None
        y1_scr, sp0_scr, sp1_scr, u_scr = refs[10:]
    ph = pl.program_id(0)
    i = pl.program_id(1)
    a = a_ref[...]
    rows = pl.ds(i * _LBLK, _LBLK)
    urows = pl.ds(i * _LBLK * B, _LBLK * B)

    @pl.when(ph == 0)
    def _():
        y1_scr[rows, :] = jnp.dot(
            a, y0_ref[...], preferred_element_type=F32).astype(BF16)

    @pl.when(ph == 1)
    def _():
        y0b = y0_ref[rows, :]
        y1b = y1_scr[rows, :]
        y2 = 2.0 * jnp.dot(a, y1_scr[...], preferred_element_type=F32) - y0b.astype(F32)
        y0r = y0b.reshape(_LBLK * B, C)
        y1r = y1b.reshape(_LBLK * B, C)
        y2r = y2.astype(BF16).reshape(_LBLK * B, C)
        g = jnp.dot(y0r, wg_ref[0], preferred_element_type=F32)
        g += jnp.dot(y1r, wg_ref[1], preferred_element_type=F32)
        g += jnp.dot(y2r, wg_ref[2], preferred_element_type=F32)
        g = jax.nn.sigmoid((g + bg_ref[...]).astype(BF16))
        cx = jnp.dot(y0r, wcx_ref[0], preferred_element_type=F32)
        cx += jnp.dot(y1r, wcx_ref[1], preferred_element_type=F32)
        cx += jnp.dot(y2r, wcx_ref[2], preferred_element_type=F32)
        r = g[:, :U]
        s0 = r * y0r[:, U:]
        sp0_scr[rows, :] = jnp.concatenate(
            [s0, cx.astype(BF16)], axis=-1).reshape(_LBLK, B * C)
        u_scr[urows, :] = g[:, U:]

    @pl.when(ph == 2)
    def _():
        sp1_scr[rows, :] = jnp.dot(
            a, sp0_scr[...], preferred_element_type=F32).astype(BF16)

    @pl.when(ph == 3)
    def _():
        sp0b = sp0_scr[rows, :]
        sp1b = sp1_scr[rows, :]
        s2 = 2.0 * jnp.dot(a, sp1_scr[...], preferred_element_type=F32) - sp0b.astype(F32)
        s0r = sp0b.reshape(_LBLK * B, C)
        s1r = sp1b.reshape(_LBLK * B, C)
        s2r = s2.astype(BF16).reshape(_LBLK * B, C)
        c = jnp.dot(s0r, wcs_ref[0], preferred_element_type=F32)
        c += jnp.dot(s1r, wcs_ref[1], preferred_element_type=F32)
        c += jnp.dot(s2r, wcs_ref[2], preferred_element_type=F32)
        cx = s0r[:, U:].astype(F32)
        c = jnp.tanh((c + cx + bc_ref[...]).astype(BF16)).astype(F32)
        u = u_scr[urows, :].astype(F32)
        y0r = y0_ref[rows, :].reshape(_LBLK * B, C)
        hx = y0r[:, U:].astype(F32)
        hn = u * hx + (1.0 - u) * c

        if y0n_ref is not None:
            h1pr = h1p_ref[...].reshape(_LBLK * B, C)
            y0n = jnp.concatenate([hn.astype(BF16), h1pr[:, U:]], axis=-1)
            y0n_ref[...] = y0n.reshape(_LBLK, B * C)

        h5 = hn.reshape(_LBLK // 128, 2, 64, B, U)
        cc = jnp.concatenate([h5[:, 0], h5[:, 1]], axis=-1)
        hbm = jnp.transpose(cc, (2, 0, 1, 3)).reshape(B, _LBLK * U)
        stack_ref[...] = hbm.reshape(1, B, _LBLK * U)
        if out1_ref is not None:
            out1_ref[...] = hbm


def _layer(adj, y0, h1p, wg, bg, wcx, wcs, bc, stack_in=None):
    is_l0 = stack_in is None

    def blk3(p_, i_):
        return (jnp.where(p_ == 3, i_, 0), 0)

    in_specs = [
        pl.BlockSpec((_LBLK, N), lambda p_, i_: (i_, 0)),
        pl.BlockSpec((N, B * C), lambda p_, i_: (0, 0)),
        pl.BlockSpec((3, C, C), lambda p_, i_: (0, 0, 0)),
        pl.BlockSpec((3, C, U), lambda p_, i_: (0, 0, 0)),
        pl.BlockSpec((3, C, U), lambda p_, i_: (0, 0, 0)),
        pl.BlockSpec((1, C), lambda p_, i_: (0, 0)),
        pl.BlockSpec((1, U), lambda p_, i_: (0, 0)),
    ]
    args = [adj, y0, wg, wcx, wcs, bg, bc]
    slot = 0 if is_l0 else 1
    stack_spec = pl.BlockSpec(
        (1, B, _LBLK * U), lambda p_, i_: (slot, 0, jnp.where(p_ == 3, i_, 0)))
    stack_shape = jax.ShapeDtypeStruct((2, B, N * U), F32)
    aliases = {}
    if is_l0:
        in_specs.append(pl.BlockSpec((_LBLK, B * C), blk3))
        args.append(h1p)
        out_specs = [pl.BlockSpec((_LBLK, B * C), blk3), stack_spec]
        out_shape = [jax.ShapeDtypeStruct((N, B * C), BF16), stack_shape]
    else:
        in_specs.append(pl.BlockSpec((1, 8, 128), lambda p_, i_: (0, 0, 0)))
        args.append(stack_in)
        aliases = {7: 0}
        out_specs = [
            stack_spec,
            pl.BlockSpec((B, _LBLK * U), lambda p_, i_: (0, jnp.where(p_ == 3, i_, 0))),
        ]
        out_shape = [stack_shape, jax.ShapeDtypeStruct((B, N * U), F32)]
    return pl.pallas_call(
        functools.partial(_layer_body, is_l0),
        grid=(4, N // _LBLK),
        in_specs=in_specs,
        out_specs=out_specs,
        out_shape=out_shape,
        input_output_aliases=aliases,
        scratch_shapes=[
            pltpu.VMEM((N, B * C), BF16),
            pltpu.VMEM((N, B * C), BF16),
            pltpu.VMEM((N, B * C), BF16),
            pltpu.VMEM((N * B, U), BF16),
        ],
        compiler_params=_ARB2,
    )(*args)


def _prep_w(w, cin):
    o = w.shape[1]
    wr = w.reshape(cin + U, 3, o).transpose(1, 0, 2)
    wx = wr[:, :cin, :]
    wh = wr[:, cin:, :]
    pad = jnp.zeros((3, U - cin, o), w.dtype)
    wxp = jnp.concatenate([wx, pad], axis=1)
    return wxp, wh


def kernel(inputs, adj, hidden_state,
           W_gate_0, b_gate_0, W_cand_0, b_cand_0,
           W_gate_1, b_gate_1, W_cand_1, b_cand_1):
    adj_bf = adj[_PIV][:, _PIV].astype(BF16)
    xin = inputs.astype(BF16).reshape(B, N, 2).transpose(1, 0, 2)[_PIV]
    xin = jnp.pad(xin, ((0, 0), (0, 0), (0, U - 2)))
    y0_l0, h1p = _prep(xin, hidden_state)

    zU = jnp.zeros((3, U, U), F32)
    wgx0, wgh0 = _prep_w(W_gate_0, 2)
    wg0 = jnp.concatenate([wgx0, wgh0], axis=1).astype(BF16)
    wcx0, wcs0 = _prep_w(W_cand_0, 2)
    wcx0 = jnp.concatenate([wcx0, zU], axis=1).astype(BF16)
    wcs0 = jnp.concatenate([wcs0, zU], axis=1).astype(BF16)
    wgx1, wgh1 = _prep_w(W_gate_1, U)
    wg1 = jnp.concatenate([wgx1, wgh1], axis=1).astype(BF16)
    wcx1, wcs1 = _prep_w(W_cand_1, U)
    wcx1 = jnp.concatenate([wcx1, zU], axis=1).astype(BF16)
    wcs1 = jnp.concatenate([wcs1, zU], axis=1).astype(BF16)
    bg0 = b_gate_0.reshape(1, C)
    bc0 = b_cand_0.reshape(1, U)
    bg1 = b_gate_1.reshape(1, C)
    bc1 = b_cand_1.reshape(1, U)

    y0_l1, stack0 = _layer(adj_bf, y0_l0, h1p, wg0, bg0, wcx0, wcs0, bc0)
    stack, out1 = _layer(adj_bf, y0_l1, h1p, wg1, bg1, wcx1, wcs1, bc1,
                         stack_in=stack0)

    return (out1, stack)


# speedup vs baseline: 1.1760x; 1.0549x over previous
import functools

import jax
import jax.numpy as jnp
import numpy as np
from jax.experimental import pallas as pl
from jax.experimental.pallas import tpu as pltpu

N = 1024
B = 32
U = 64
C = 2 * U
BLK = 256
F32 = jnp.float32
BF16 = jnp.bfloat16

_ARB2 = pltpu.CompilerParams(dimension_semantics=("arbitrary", "arbitrary"))

_PIV = np.arange(N).reshape(N // 128, 64, 2).transpose(0, 2, 1).reshape(-1)


def _unpack_nm(h2, blkp):
    v = h2.astype(BF16).reshape(B, blkp // 2, 2 * U)
    t = jnp.transpose(v, (1, 0, 2))
    return jnp.concatenate([t[:, :, :U], t[:, :, U:]], axis=0)


def _rowperm(a):
    v = a.reshape((64, 2) + a.shape[1:])
    return jnp.concatenate([v[:, 0], v[:, 1]], axis=0)


def _prep_body(adj_ref, pmat_ref, x_ref, hs_ref, adjp_ref, y0_ref, h1p_ref):
    ph = pl.program_id(0)

    @pl.when(ph == 0)
    def _():
        ar = _rowperm(adj_ref[...]).astype(BF16)
        adjp_ref[...] = jnp.dot(
            ar, pmat_ref[...], preferred_element_type=F32).astype(BF16)

    @pl.when(ph == 1)
    def _():
        x = _rowperm(x_ref[...])
        blkp = x.shape[0]
        hs = hs_ref[...]
        h0 = _unpack_nm(hs[0], blkp)
        h1 = _unpack_nm(hs[1], blkp)
        y0 = jnp.concatenate([x, h0], axis=-1)
        y0_ref[...] = y0.reshape(blkp, B * C)
        h1p = jnp.concatenate([jnp.zeros_like(x), h1], axis=-1)
        h1p_ref[...] = h1p.reshape(blkp, B * C)


def _prep(adj, pmat, xin, hs):
    blkp = 128

    def blk0(p_, i_):
        return (jnp.where(p_ == 0, i_, 0), 0)

    def blk1(p_, i_):
        return (jnp.where(p_ == 1, i_, 0), 0)

    return pl.pallas_call(
        _prep_body,
        grid=(2, N // blkp),
        in_specs=[
            pl.BlockSpec((blkp, N), blk0),
            pl.BlockSpec((N, N), lambda p_, i_: (0, 0)),
            pl.BlockSpec((blkp, B, U), lambda p_, i_: (jnp.where(p_ == 1, i_, 0), 0, 0)),
            pl.BlockSpec((2, B, blkp * U), lambda p_, i_: (0, 0, jnp.where(p_ == 1, i_, 0))),
        ],
        out_specs=[
            pl.BlockSpec((blkp, N), blk0),
            pl.BlockSpec((blkp, B * C), blk1),
            pl.BlockSpec((blkp, B * C), blk1),
        ],
        out_shape=[
            jax.ShapeDtypeStruct((N, N), BF16),
            jax.ShapeDtypeStruct((N, B * C), BF16),
            jax.ShapeDtypeStruct((N, B * C), BF16),
        ],
        compiler_params=_ARB2,
    )(adj, pmat, xin, hs)


_LBLK = 128


def _layer_body(is_l0, *refs):
    (a_ref, y0_ref, wg_ref, wcx_ref, wcs_ref, bg_ref, bc_ref) = refs[:7]
    if is_l0:
        h1p_ref, y0n_ref, stack_ref = refs[7:10]
        out1_ref = None
        y1_scr, sp0_scr, sp1_scr, u_scr = refs[10:]
    else:
        _, stack_ref, out1_ref = refs[7:10]
        h1p_ref = y0n_ref = None
        y1_scr, sp0_scr, sp1_scr, u_scr = refs[10:]
    ph = pl.program_id(0)
    i = pl.program_id(1)
    a = a_ref[...]
    rows = pl.ds(i * _LBLK, _LBLK)
    urows = pl.ds(i * _LBLK * B, _LBLK * B)

    @pl.when(ph == 0)
    def _():
        y1_scr[rows, :] = jnp.dot(
            a, y0_ref[...], preferred_element_type=F32).astype(BF16)

    @pl.when(ph == 1)
    def _():
        y0b = y0_ref[rows, :]
        y1b = y1_scr[rows, :]
        y2 = 2.0 * jnp.dot(a, y1_scr[...], preferred_element_type=F32) - y0b.astype(F32)
        y0r = y0b.reshape(_LBLK * B, C)
        y1r = y1b.reshape(_LBLK * B, C)
        y2r = y2.astype(BF16).reshape(_LBLK * B, C)
        g = jnp.dot(y0r, wg_ref[0], preferred_element_type=F32)
        g += jnp.dot(y1r, wg_ref[1], preferred_element_type=F32)
        g += jnp.dot(y2r, wg_ref[2], preferred_element_type=F32)
        g = jax.nn.sigmoid((g + bg_ref[...]).astype(BF16))
        cx = jnp.dot(y0r, wcx_ref[0], preferred_element_type=F32)
        cx += jnp.dot(y1r, wcx_ref[1], preferred_element_type=F32)
        cx += jnp.dot(y2r, wcx_ref[2], preferred_element_type=F32)
        r = g[:, :U]
        s0 = r * y0r[:, U:]
        sp0_scr[rows, :] = jnp.concatenate(
            [s0, cx.astype(BF16)], axis=-1).reshape(_LBLK, B * C)
        u_scr[urows, :] = g[:, U:]

    @pl.when(ph == 2)
    def _():
        sp1_scr[rows, :] = jnp.dot(
            a, sp0_scr[...], preferred_element_type=F32).astype(BF16)

    @pl.when(ph == 3)
    def _():
        sp0b = sp0_scr[rows, :]
        sp1b = sp1_scr[rows, :]
        s2 = 2.0 * jnp.dot(a, sp1_scr[...], preferred_element_type=F32) - sp0b.astype(F32)
        s0r = sp0b.reshape(_LBLK * B, C)
        s1r = sp1b.reshape(_LBLK * B, C)
        s2r = s2.astype(BF16).reshape(_LBLK * B, C)
        c = jnp.dot(s0r, wcs_ref[0], preferred_element_type=F32)
        c += jnp.dot(s1r, wcs_ref[1], preferred_element_type=F32)
        c += jnp.dot(s2r, wcs_ref[2], preferred_element_type=F32)
        cx = s0r[:, U:].astype(F32)
        c = jnp.tanh((c + cx + bc_ref[...]).astype(BF16)).astype(F32)
        u = u_scr[urows, :].astype(F32)
        y0r = y0_ref[rows, :].reshape(_LBLK * B, C)
        hx = y0r[:, U:].astype(F32)
        hn = u * hx + (1.0 - u) * c

        if y0n_ref is not None:
            h1pr = h1p_ref[...].reshape(_LBLK * B, C)
            y0n = jnp.concatenate([hn.astype(BF16), h1pr[:, U:]], axis=-1)
            y0n_ref[...] = y0n.reshape(_LBLK, B * C)

        h5 = hn.reshape(_LBLK // 128, 2, 64, B, U)
        cc = jnp.concatenate([h5[:, 0], h5[:, 1]], axis=-1)
        hbm = jnp.transpose(cc, (2, 0, 1, 3)).reshape(B, _LBLK * U)
        stack_ref[...] = hbm.reshape(1, B, _LBLK * U)
        if out1_ref is not None:
            out1_ref[...] = hbm


def _layer(adj, y0, h1p, wg, bg, wcx, wcs, bc, stack_in=None):
    is_l0 = stack_in is None

    def blk3(p_, i_):
        return (jnp.where(p_ == 3, i_, 0), 0)

    in_specs = [
        pl.BlockSpec((_LBLK, N), lambda p_, i_: (i_, 0)),
        pl.BlockSpec((N, B * C), lambda p_, i_: (0, 0)),
        pl.BlockSpec((3, C, C), lambda p_, i_: (0, 0, 0)),
        pl.BlockSpec((3, C, U), lambda p_, i_: (0, 0, 0)),
        pl.BlockSpec((3, C, U), lambda p_, i_: (0, 0, 0)),
        pl.BlockSpec((1, C), lambda p_, i_: (0, 0)),
        pl.BlockSpec((1, U), lambda p_, i_: (0, 0)),
    ]
    args = [adj, y0, wg, wcx, wcs, bg, bc]
    slot = 0 if is_l0 else 1
    stack_spec = pl.BlockSpec(
        (1, B, _LBLK * U), lambda p_, i_: (slot, 0, jnp.where(p_ == 3, i_, 0)))
    stack_shape = jax.ShapeDtypeStruct((2, B, N * U), F32)
    aliases = {}
    if is_l0:
        in_specs.append(pl.BlockSpec((_LBLK, B * C), blk3))
        args.append(h1p)
        out_specs = [pl.BlockSpec((_LBLK, B * C), blk3), stack_spec]
        out_shape = [jax.ShapeDtypeStruct((N, B * C), BF16), stack_shape]
    else:
        in_specs.append(pl.BlockSpec((1, 8, 128), lambda p_, i_: (0, 0, 0)))
        args.append(stack_in)
        aliases = {7: 0}
        out_specs = [
            stack_spec,
            pl.BlockSpec((B, _LBLK * U), lambda p_, i_: (0, jnp.where(p_ == 3, i_, 0))),
        ]
        out_shape = [stack_shape, jax.ShapeDtypeStruct((B, N * U), F32)]
    return pl.pallas_call(
        functools.partial(_layer_body, is_l0),
        grid=(4, N // _LBLK),
        in_specs=in_specs,
        out_specs=out_specs,
        out_shape=out_shape,
        input_output_aliases=aliases,
        scratch_shapes=[
            pltpu.VMEM((N, B * C), BF16),
            pltpu.VMEM((N, B * C), BF16),
            pltpu.VMEM((N, B * C), BF16),
            pltpu.VMEM((N * B, U), BF16),
        ],
        compiler_params=_ARB2,
    )(*args)


def _prep_w(w, cin):
    o = w.shape[1]
    wr = w.reshape(cin + U, 3, o).transpose(1, 0, 2)
    wx = wr[:, :cin, :]
    wh = wr[:, cin:, :]
    pad = jnp.zeros((3, U - cin, o), w.dtype)
    wxp = jnp.concatenate([wx, pad], axis=1)
    return wxp, wh


def kernel(inputs, adj, hidden_state,
           W_gate_0, b_gate_0, W_cand_0, b_cand_0,
           W_gate_1, b_gate_1, W_cand_1, b_cand_1):
    xin = inputs.astype(BF16).reshape(B, N, 2).transpose(1, 0, 2)
    xin = jnp.pad(xin, ((0, 0), (0, 0), (0, U - 2)))
    pmat = jnp.asarray(np.eye(N, dtype=np.float32)[:, _PIV], dtype=BF16)
    adj_bf, y0_l0, h1p = _prep(adj, pmat, xin, hidden_state)

    zU = jnp.zeros((3, U, U), F32)
    wgx0, wgh0 = _prep_w(W_gate_0, 2)
    wg0 = jnp.concatenate([wgx0, wgh0], axis=1).astype(BF16)
    wcx0, wcs0 = _prep_w(W_cand_0, 2)
    wcx0 = jnp.concatenate([wcx0, zU], axis=1).astype(BF16)
    wcs0 = jnp.concatenate([wcs0, zU], axis=1).astype(BF16)
    wgx1, wgh1 = _prep_w(W_gate_1, U)
    wg1 = jnp.concatenate([wgx1, wgh1], axis=1).astype(BF16)
    wcx1, wcs1 = _prep_w(W_cand_1, U)
    wcx1 = jnp.concatenate([wcx1, zU], axis=1).astype(BF16)
    wcs1 = jnp.concatenate([wcs1, zU], axis=1).astype(BF16)
    bg0 = b_gate_0.reshape(1, C)
    bc0 = b_cand_0.reshape(1, U)
    bg1 = b_gate_1.reshape(1, C)
    bc1 = b_cand_1.reshape(1, U)

    y0_l1, stack0 = _layer(adj_bf, y0_l0, h1p, wg0, bg0, wcx0, wcs0, bc0)
    stack, out1 = _layer(adj_bf, y0_l1, h1p, wg1, bg1, wcx1, wcs1, bc1,
                         stack_in=stack0)

    return (out1, stack)
